# baseline (speedup 1.0000x reference)
import jax
import jax.numpy as jnp
from jax import lax
from jax.experimental import pallas as pl
from jax.experimental.pallas import tpu as pltpu
from jax.experimental.pallas import tpu_sc as plsc

N_PICKS = 10
IOU_THRESH = 0.25
NEG_INF = float("-inf")
L = 16
NS = 16
N = 20000
CHUNK = 1264
NVEC = CHUNK // L
TAIL = N - (NS - 1) * CHUNK
TVEC = TAIL // L


def _permute(x, idx):
    return x.at[idx].get(mode="promise_in_bounds")


def _lane_argmax(v, i, lane):
    for sh in (8, 4, 2, 1):
        pidx = lane ^ sh
        pv = _permute(v, pidx)
        pi = _permute(i, pidx)
        take = (pv > v) | ((pv == v) & (pi < i))
        v = jnp.where(take, pv, v)
        i = jnp.where(take, pi, i)
    return v, i


def _nms_body(s_hbm, b_hbm, out_hbm,
              s_ref, y0_ref, x0_ref, y1_ref, x1_ref, bb_ref,
              pub_ref, lb_ref, out_ref, shared_ref, sem):
    sid = lax.axis_index("s")
    base = pl.multiple_of(sid * CHUNK, CHUNK)
    lane = lax.broadcasted_iota(jnp.int32, (L,), 0)
    neg_inf_v = jnp.full((L,), NEG_INF, jnp.float32)
    zero_v = jnp.zeros((L,), jnp.float32)
    zero_i = jnp.zeros((L,), jnp.int32)

    def load(n):
        copies = [pltpu.make_async_copy(s_hbm.at[pl.ds(base, n)],
                                        s_ref.at[pl.ds(0, n)], sem),
                  pltpu.make_async_copy(b_hbm.at[pl.ds(base * 4, n * 4)],
                                        bb_ref.at[pl.ds(0, n * 4)], sem)]
        for c in copies:
            c.start()
        for c in copies:
            c.wait()

    @pl.when(sid < NS - 1)
    def _():
        load(CHUNK)

    @pl.when(sid == NS - 1)
    def _():
        load(TAIL)
        for i in range(TVEC, NVEC):
            s_ref[pl.ds(i * L, L)] = neg_inf_v

    def tr_body(i, carry):
        off = pl.multiple_of(i * L, L)
        idx = i * (L * 4) + lane * 4
        y0_ref[pl.ds(off, L)] = plsc.load_gather(bb_ref, [idx])
        x0_ref[pl.ds(off, L)] = plsc.load_gather(bb_ref, [idx + 1])
        y1_ref[pl.ds(off, L)] = plsc.load_gather(bb_ref, [idx + 2])
        x1_ref[pl.ds(off, L)] = plsc.load_gather(bb_ref, [idx + 3])
        return carry

    lax.fori_loop(0, NVEC, tr_body, 0)

    def amax_body(i, carry):
        bv, bc = carry
        off = pl.multiple_of(i * L, L)
        v = s_ref[pl.ds(off, L)]
        cond = v > bv
        return jnp.where(cond, v, bv), jnp.where(cond, i, bc)

    bv, bc = lax.fori_loop(0, NVEC, amax_body, (neg_inf_v, zero_i))

    w0m = w0row = None
    for t in range(N_PICKS):
        m_v, li_v = _lane_argmax(bv, bc * L + lane, lane)
        row = jnp.where(lane == 0, m_v, zero_v)
        row = jnp.where(lane == 1, plsc.load_gather(y0_ref, [li_v]), row)
        row = jnp.where(lane == 2, plsc.load_gather(x0_ref, [li_v]), row)
        row = jnp.where(lane == 3, plsc.load_gather(y1_ref, [li_v]), row)
        row = jnp.where(lane == 4, plsc.load_gather(x1_ref, [li_v]), row)

        pub_ref[pl.ds(0, L)] = row
        slot = pl.multiple_of(t * NS * L + sid * L, L)
        pltpu.sync_copy(pub_ref, shared_ref.at[pl.ds(slot, L)])
        plsc.subcore_barrier()

        pltpu.sync_copy(shared_ref.at[pl.ds(t * NS * L, NS * L)], lb_ref)
        bm, brow = neg_inf_v, zero_v
        for r in range(NS):
            rrow = lb_ref[pl.ds(r * L, L)]
            rm = _permute(rrow, lane * 0)
            cond = rm > bm
            bm = jnp.where(cond, rm, bm)
            brow = jnp.where(cond, rrow, brow)
        if t == 0:
            w0m, w0row = bm, brow
        else:
            fb = bm == neg_inf_v
            bm = jnp.where(fb, w0m, bm)
            brow = jnp.where(fb, w0row, brow)
        plsc.store_scatter(out_ref, [lane * 0 + (t * 5) + lane], brow,
                           mask=lane < 5)

        if t == N_PICKS - 1:
            break

        by0 = _permute(brow, lane * 0 + 1)
        bx0 = _permute(brow, lane * 0 + 2)
        by1 = _permute(brow, lane * 0 + 3)
        bx1 = _permute(brow, lane * 0 + 4)
        carea = (by1 - by0) * (bx1 - bx0)

        def sup_body(i, carry):
            nbv, nbc = carry
            off = pl.multiple_of(i * L, L)
            vy0 = y0_ref[pl.ds(off, L)]
            vx0 = x0_ref[pl.ds(off, L)]
            vy1 = y1_ref[pl.ds(off, L)]
            vx1 = x1_ref[pl.ds(off, L)]
            vs = s_ref[pl.ds(off, L)]
            l0 = jnp.minimum(vy1, by1) - jnp.maximum(vy0, by0)
            l1 = jnp.minimum(vx1, bx1) - jnp.maximum(vx0, bx0)
            inter = jnp.where((l0 < 0) | (l1 < 0), 0.0, l0 * l1)
            area = (vy1 - vy0) * (vx1 - vx0)
            iou = inter / (area + carea - inter)
            ns = jnp.where(iou < IOU_THRESH, vs, neg_inf_v)
            s_ref[pl.ds(off, L)] = ns
            cond = ns > nbv
            return jnp.where(cond, ns, nbv), jnp.where(cond, i, nbc)

        bv, bc = lax.fori_loop(0, NVEC, sup_body, (neg_inf_v, zero_i))

    @pl.when(sid == 0)
    def _():
        pltpu.sync_copy(out_ref, out_hbm)


_nms_call = pl.kernel(
    _nms_body,
    out_type=jax.ShapeDtypeStruct((N_PICKS * 5,), jnp.float32),
    mesh=plsc.VectorSubcoreMesh(core_axis_name="c", subcore_axis_name="s",
                                num_cores=1),
    compiler_params=pltpu.CompilerParams(needs_layout_passes=False,
                                         use_tc_tiling_on_sc=False),
    scratch_types=[
        pltpu.VMEM((CHUNK,), jnp.float32),
        pltpu.VMEM((CHUNK,), jnp.float32),
        pltpu.VMEM((CHUNK,), jnp.float32),
        pltpu.VMEM((CHUNK,), jnp.float32),
        pltpu.VMEM((CHUNK,), jnp.float32),
        pltpu.VMEM((CHUNK * 4,), jnp.float32),
        pltpu.VMEM((L,), jnp.float32),
        pltpu.VMEM((NS * L,), jnp.float32),
        pltpu.VMEM((N_PICKS * 5,), jnp.float32),
        pltpu.VMEM_SHARED((N_PICKS * NS * L,), jnp.float32),
        pltpu.SemaphoreType.DMA,
    ],
)


def kernel(scores, boxes, top_n):
    del top_n
    return _nms_call(scores, boxes.reshape(-1)).reshape(N_PICKS, 5)


# speedup vs baseline: 1.7451x; 1.7451x over previous
import jax
import jax.numpy as jnp
from jax import lax
from jax.experimental import pallas as pl
from jax.experimental.pallas import tpu as pltpu
from jax.experimental.pallas import tpu_sc as plsc

N_PICKS = 10
IOU_THRESH = 0.25
NEG_INF = float("-inf")
L = 16
NS = 16
CHUNK = 1280
NVEC = CHUNK // L
UNROLL = 4
NPAD = NS * CHUNK


def _permute(x, idx):
    return x.at[idx].get(mode="promise_in_bounds")


def _lane_argmax(v, i, lane):
    for sh in (8, 4, 2, 1):
        pidx = lane ^ sh
        pv = _permute(v, pidx)
        pi = _permute(i, pidx)
        take = (pv > v) | ((pv == v) & (pi < i))
        v = jnp.where(take, pv, v)
        i = jnp.where(take, pi, i)
    return v, i


def _nms_body(s_hbm, y0_hbm, x0_hbm, y1_hbm, x1_hbm, out_hbm,
              s_ref, y0_ref, x0_ref, y1_ref, x1_ref,
              pub_ref, lb_ref, out_ref, shared_ref, sem):
    sid = lax.axis_index("s")
    base = pl.multiple_of(sid * CHUNK, CHUNK)

    copies = [pltpu.make_async_copy(h.at[pl.ds(base, CHUNK)], r, sem)
              for h, r in ((s_hbm, s_ref), (y0_hbm, y0_ref),
                           (x0_hbm, x0_ref), (y1_hbm, y1_ref),
                           (x1_hbm, x1_ref))]
    for c in copies:
        c.start()
    for c in copies:
        c.wait()

    lane = lax.broadcasted_iota(jnp.int32, (L,), 0)
    neg_inf_v = jnp.full((L,), NEG_INF, jnp.float32)
    zero_v = jnp.zeros((L,), jnp.float32)
    zero_i = jnp.zeros((L,), jnp.int32)

    def amax_body(j, carry):
        bv, bc = carry
        for u in range(UNROLL):
            i = j * UNROLL + u
            off = pl.multiple_of(i * L, L)
            v = s_ref[pl.ds(off, L)]
            cond = v > bv
            bv = jnp.where(cond, v, bv)
            bc = jnp.where(cond, i, bc)
        return bv, bc

    bv, bc = lax.fori_loop(0, NVEC // UNROLL, amax_body, (neg_inf_v, zero_i))

    w0m = w0row = None
    for t in range(N_PICKS):
        m_v, li_v = _lane_argmax(bv, bc * L + lane, lane)
        row = jnp.where(lane == 0, m_v, zero_v)
        row = jnp.where(lane == 1, plsc.load_gather(y0_ref, [li_v]), row)
        row = jnp.where(lane == 2, plsc.load_gather(x0_ref, [li_v]), row)
        row = jnp.where(lane == 3, plsc.load_gather(y1_ref, [li_v]), row)
        row = jnp.where(lane == 4, plsc.load_gather(x1_ref, [li_v]), row)

        pub_ref[pl.ds(0, L)] = row
        slot = pl.multiple_of(t * NS * L + sid * L, L)
        pltpu.sync_copy(pub_ref, shared_ref.at[pl.ds(slot, L)])
        plsc.subcore_barrier()

        pltpu.sync_copy(shared_ref.at[pl.ds(t * NS * L, NS * L)], lb_ref)
        bm, brow = neg_inf_v, zero_v
        for r in range(NS):
            rrow = lb_ref[pl.ds(r * L, L)]
            rm = _permute(rrow, lane * 0)
            cond = rm > bm
            bm = jnp.where(cond, rm, bm)
            brow = jnp.where(cond, rrow, brow)
        if t == 0:
            w0m, w0row = bm, brow
        else:
            fb = bm == neg_inf_v
            bm = jnp.where(fb, w0m, bm)
            brow = jnp.where(fb, w0row, brow)
        plsc.store_scatter(out_ref, [lane * 0 + (t * 5) + lane], brow,
                           mask=lane < 5)

        if t == N_PICKS - 1:
            break

        by0 = _permute(brow, lane * 0 + 1)
        bx0 = _permute(brow, lane * 0 + 2)
        by1 = _permute(brow, lane * 0 + 3)
        bx1 = _permute(brow, lane * 0 + 4)
        carea = (by1 - by0) * (bx1 - bx0)

        def sup_body(j, carry):
            nbv, nbc = carry
            for u in range(UNROLL):
                i = j * UNROLL + u
                off = pl.multiple_of(i * L, L)
                vy0 = y0_ref[pl.ds(off, L)]
                vx0 = x0_ref[pl.ds(off, L)]
                vy1 = y1_ref[pl.ds(off, L)]
                vx1 = x1_ref[pl.ds(off, L)]
                vs = s_ref[pl.ds(off, L)]
                l0 = jnp.minimum(vy1, by1) - jnp.maximum(vy0, by0)
                l1 = jnp.minimum(vx1, bx1) - jnp.maximum(vx0, bx0)
                inter = jnp.where((l0 < 0) | (l1 < 0), 0.0, l0 * l1)
                area = (vy1 - vy0) * (vx1 - vx0)
                iou = inter / (area + carea - inter)
                ns = jnp.where(iou < IOU_THRESH, vs, neg_inf_v)
                s_ref[pl.ds(off, L)] = ns
                cond = ns > nbv
                nbv = jnp.where(cond, ns, nbv)
                nbc = jnp.where(cond, i, nbc)
            return nbv, nbc

        bv, bc = lax.fori_loop(0, NVEC // UNROLL, sup_body,
                               (neg_inf_v, zero_i))

    @pl.when(sid == 0)
    def _():
        pltpu.sync_copy(out_ref, out_hbm)


_nms_call = pl.kernel(
    _nms_body,
    out_type=jax.ShapeDtypeStruct((N_PICKS * 5,), jnp.float32),
    mesh=plsc.VectorSubcoreMesh(core_axis_name="c", subcore_axis_name="s",
                                num_cores=1),
    compiler_params=pltpu.CompilerParams(needs_layout_passes=False),
    scratch_types=[
        pltpu.VMEM((CHUNK,), jnp.float32),
        pltpu.VMEM((CHUNK,), jnp.float32),
        pltpu.VMEM((CHUNK,), jnp.float32),
        pltpu.VMEM((CHUNK,), jnp.float32),
        pltpu.VMEM((CHUNK,), jnp.float32),
        pltpu.VMEM((L,), jnp.float32),
        pltpu.VMEM((NS * L,), jnp.float32),
        pltpu.VMEM((N_PICKS * 5,), jnp.float32),
        pltpu.VMEM_SHARED((N_PICKS * NS * L,), jnp.float32),
        pltpu.SemaphoreType.DMA,
    ],
)


def kernel(scores, boxes, top_n):
    del top_n
    n = scores.shape[0]
    pad = NPAD - n
    s = jnp.concatenate([scores, jnp.full((pad,), NEG_INF, jnp.float32)])
    b = jnp.concatenate([boxes, jnp.zeros((pad, 4), jnp.float32)], axis=0)
    out = _nms_call(s, b[:, 0], b[:, 1], b[:, 2], b[:, 3])
    return out.reshape(N_PICKS, 5)


# speedup vs baseline: 1.9876x; 1.1390x over previous
import jax
import jax.numpy as jnp
from jax import lax
from jax.experimental import pallas as pl
from jax.experimental.pallas import tpu as pltpu
from jax.experimental.pallas import tpu_sc as plsc

N_PICKS = 10
IOU_THRESH = 0.25
NEG_INF = float("-inf")
L = 16
NS = 16
CHUNK = 1280
NVEC = CHUNK // L
UNROLL = 4
NPAD = NS * CHUNK


def _permute(x, idx):
    return x.at[idx].get(mode="promise_in_bounds")


def _lane_argmax(v, i, lane):
    for sh in (8, 4, 2, 1):
        pidx = lane ^ sh
        pv = _permute(v, pidx)
        pi = _permute(i, pidx)
        take = (pv > v) | ((pv == v) & (pi < i))
        v = jnp.where(take, pv, v)
        i = jnp.where(take, pi, i)
    return v, i


def _nms_body(s_hbm, y0_hbm, x0_hbm, y1_hbm, x1_hbm, out_hbm,
              s_ref, y0_ref, x0_ref, y1_ref, x1_ref,
              pub_ref, lb_ref, out_ref, shared_ref, sem):
    sid = lax.axis_index("s")
    base = pl.multiple_of(sid * CHUNK, CHUNK)

    copies = [pltpu.make_async_copy(h.at[pl.ds(base, CHUNK)], r, sem)
              for h, r in ((s_hbm, s_ref), (y0_hbm, y0_ref),
                           (x0_hbm, x0_ref), (y1_hbm, y1_ref),
                           (x1_hbm, x1_ref))]
    for c in copies:
        c.start()
    for c in copies:
        c.wait()

    lane = lax.broadcasted_iota(jnp.int32, (L,), 0)
    neg_inf_v = jnp.full((L,), NEG_INF, jnp.float32)
    zero_v = jnp.zeros((L,), jnp.float32)
    zero_i = jnp.zeros((L,), jnp.int32)

    def amax_body(j, carry):
        bv, bc = carry
        for u in range(UNROLL):
            i = j * UNROLL + u
            off = pl.multiple_of(i * L, L)
            v = s_ref[pl.ds(off, L)]
            cond = v > bv
            bv = jnp.where(cond, v, bv)
            bc = jnp.where(cond, i, bc)
        return bv, bc

    bv0, bc0 = lax.fori_loop(0, NVEC // UNROLL, amax_body,
                             (neg_inf_v, zero_i))

    def pick_body(t, carry):
        bv, bc, w0m, w0row = carry

        m_v, li_v = _lane_argmax(bv, bc * L + lane, lane)
        row = jnp.where(lane == 0, m_v, zero_v)
        row = jnp.where(lane == 1, plsc.load_gather(y0_ref, [li_v]), row)
        row = jnp.where(lane == 2, plsc.load_gather(x0_ref, [li_v]), row)
        row = jnp.where(lane == 3, plsc.load_gather(y1_ref, [li_v]), row)
        row = jnp.where(lane == 4, plsc.load_gather(x1_ref, [li_v]), row)

        pub_ref[pl.ds(0, L)] = row
        slot = pl.multiple_of(t * (NS * L) + sid * L, L)
        pltpu.sync_copy(pub_ref, shared_ref.at[pl.ds(slot, L)])
        plsc.subcore_barrier()

        rb = pl.multiple_of(t * (NS * L), L)
        pltpu.sync_copy(shared_ref.at[pl.ds(rb, NS * L)], lb_ref)

        def red_body(r, carry2):
            bm, brow = carry2
            roff = pl.multiple_of(r * L, L)
            rrow = lb_ref[pl.ds(roff, L)]
            rm = _permute(rrow, lane * 0)
            cond = rm > bm
            return jnp.where(cond, rm, bm), jnp.where(cond, rrow, brow)

        bm, brow = lax.fori_loop(0, NS, red_body, (neg_inf_v, zero_v))

        fb = bm == neg_inf_v
        bm = jnp.where(fb, w0m, bm)
        brow = jnp.where(fb, w0row, brow)
        first = jnp.broadcast_to(t == 0, (L,))
        w0m = jnp.where(first, bm, w0m)
        w0row = jnp.where(first, brow, w0row)

        plsc.store_scatter(out_ref, [t * 5 + lane], brow, mask=lane < 5)

        by0 = _permute(brow, lane * 0 + 1)
        bx0 = _permute(brow, lane * 0 + 2)
        by1 = _permute(brow, lane * 0 + 3)
        bx1 = _permute(brow, lane * 0 + 4)
        carea = (by1 - by0) * (bx1 - bx0)

        def sup_body(j, carry2):
            nbv, nbc = carry2
            for u in range(UNROLL):
                i = j * UNROLL + u
                off = pl.multiple_of(i * L, L)
                vy0 = y0_ref[pl.ds(off, L)]
                vx0 = x0_ref[pl.ds(off, L)]
                vy1 = y1_ref[pl.ds(off, L)]
                vx1 = x1_ref[pl.ds(off, L)]
                vs = s_ref[pl.ds(off, L)]
                l0 = jnp.minimum(vy1, by1) - jnp.maximum(vy0, by0)
                l1 = jnp.minimum(vx1, bx1) - jnp.maximum(vx0, bx0)
                inter = jnp.where((l0 < 0) | (l1 < 0), 0.0, l0 * l1)
                area = (vy1 - vy0) * (vx1 - vx0)
                iou = inter / (area + carea - inter)
                ns = jnp.where(iou < IOU_THRESH, vs, neg_inf_v)
                s_ref[pl.ds(off, L)] = ns
                cond = ns > nbv
                nbv = jnp.where(cond, ns, nbv)
                nbc = jnp.where(cond, i, nbc)
            return nbv, nbc

        trips = jnp.where(t == N_PICKS - 1, 0, NVEC // UNROLL)
        bv, bc = lax.fori_loop(0, trips, sup_body, (neg_inf_v, zero_i))
        return bv, bc, w0m, w0row

    lax.fori_loop(0, N_PICKS, pick_body, (bv0, bc0, zero_v, zero_v))

    @pl.when(sid == 0)
    def _():
        pltpu.sync_copy(out_ref, out_hbm)


_nms_call = pl.kernel(
    _nms_body,
    out_type=jax.ShapeDtypeStruct((N_PICKS * 5,), jnp.float32),
    mesh=plsc.VectorSubcoreMesh(core_axis_name="c", subcore_axis_name="s",
                                num_cores=1),
    compiler_params=pltpu.CompilerParams(needs_layout_passes=False),
    scratch_types=[
        pltpu.VMEM((CHUNK,), jnp.float32),
        pltpu.VMEM((CHUNK,), jnp.float32),
        pltpu.VMEM((CHUNK,), jnp.float32),
        pltpu.VMEM((CHUNK,), jnp.float32),
        pltpu.VMEM((CHUNK,), jnp.float32),
        pltpu.VMEM((L,), jnp.float32),
        pltpu.VMEM((NS * L,), jnp.float32),
        pltpu.VMEM((N_PICKS * 5,), jnp.float32),
        pltpu.VMEM_SHARED((N_PICKS * NS * L,), jnp.float32),
        pltpu.SemaphoreType.DMA,
    ],
)


def kernel(scores, boxes, top_n):
    del top_n
    n = scores.shape[0]
    pad = NPAD - n
    s = jnp.concatenate([scores, jnp.full((pad,), NEG_INF, jnp.float32)])
    b = jnp.concatenate([boxes, jnp.zeros((pad, 4), jnp.float32)], axis=0)
    out = _nms_call(s, b[:, 0], b[:, 1], b[:, 2], b[:, 3])
    return out.reshape(N_PICKS, 5)
